# Initial kernel scaffold; baseline (speedup 1.0000x reference)
import jax
import jax.numpy as jnp
from jax.experimental import pallas as pl


def kernel(obj_vecs, pred_vecs, edges, W1a, b1a, W1b, b1b, W2a, b2a, W2b, b2b):
    raise NotImplementedError("write your pallas kernel here")


# speedup vs baseline: 87.4398x; 87.4398x over previous
import functools

import jax
import jax.numpy as jnp
from jax import lax
from jax.experimental import pallas as pl
from jax.experimental.pallas import tpu as pltpu
from jax.experimental.pallas import tpu_sc as plsc

_O, _T, _D, _H, _DOUT = 10000, 320000, 128, 512, 128

_NC, _NS = 2, 16
_NW = _NC * _NS

_EPW = _T // _NW
_GCH = 400
_GN = _EPW // _GCH

_sc_mesh = plsc.VectorSubcoreMesh(core_axis_name="c", subcore_axis_name="s")


@functools.partial(
    pl.kernel,
    mesh=_sc_mesh,
    out_type=[
        jax.ShapeDtypeStruct((_T, _D), jnp.float32),
        jax.ShapeDtypeStruct((_T, _D), jnp.float32),
    ],
    scratch_types=[
        pltpu.VMEM((_GCH,), jnp.int32),
        pltpu.VMEM((_GCH,), jnp.int32),
        pltpu.VMEM((_GCH, _D), jnp.float32),
        pltpu.VMEM((_GCH, _D), jnp.float32),
        pltpu.SemaphoreType.DMA,
        pltpu.SemaphoreType.DMA,
    ],
)
def _gather_sc(obj_hbm, sidx_hbm, oidx_hbm, srows_hbm, orows_hbm,
               sidx_v, oidx_v, srow_v, orow_v, sem_s, sem_o):
    i32 = jnp.int32
    wid = lax.axis_index("s") * i32(_NC) + lax.axis_index("c")
    base = wid * i32(_EPW)

    def body(c, carry):
        off = pl.multiple_of(base + c * i32(_GCH), 8)
        pltpu.sync_copy(sidx_hbm.at[pl.ds(off, _GCH)], sidx_v)
        pltpu.sync_copy(oidx_hbm.at[pl.ds(off, _GCH)], oidx_v)
        cs = pltpu.async_copy(obj_hbm.at[sidx_v], srow_v, sem_s)
        co = pltpu.async_copy(obj_hbm.at[oidx_v], orow_v, sem_o)
        cs.wait()
        co.wait()
        pltpu.sync_copy(srow_v, srows_hbm.at[pl.ds(off, _GCH)])
        pltpu.sync_copy(orow_v, orows_hbm.at[pl.ds(off, _GCH)])
        return carry

    lax.fori_loop(jnp.int32(0), jnp.int32(_GN), body, jnp.int32(0))


_OP = 10240
_EPT = _T // _NS
_SCH = 80
_SN = _EPT // _SCH
_CG = 128
_RPT = _OP // _NS
_ZR = 80


@functools.partial(
    pl.kernel,
    mesh=_sc_mesh,
    out_type=[
        jax.ShapeDtypeStruct((_OP, _H), jnp.float32),
        jax.ShapeDtypeStruct((_OP, _CG), jnp.float32),
    ],
    scratch_types=[
        pltpu.VMEM_SHARED((_OP, _CG), jnp.float32),
        pltpu.VMEM((_SCH,), jnp.int32),
        pltpu.VMEM((_SCH,), jnp.int32),
        pltpu.VMEM((_SCH, _CG), jnp.float32),
        pltpu.VMEM((_SCH, _CG), jnp.float32),
        pltpu.VMEM((_SCH, _CG), jnp.float32),
        pltpu.VMEM((_ZR, _CG), jnp.float32),
    ],
)
def _scatter_sc(ns_hbm, no_hbm, sidx_hbm, oidx_hbm, pooled_hbm, cnt_hbm,
                acc_sh, sidx_v, oidx_v, srow_v, orow_v, ones_v, zrow_v):
    i32 = jnp.int32
    cid = lax.axis_index("c")
    sid = lax.axis_index("s")
    rbase = sid * i32(_RPT)

    def initz(i, carry):
        r = i // i32(_CG // 16)
        k = i % i32(_CG // 16)
        zrow_v[r, pl.ds(k * i32(16), 16)] = jnp.zeros((16,), jnp.float32)
        return carry

    lax.fori_loop(jnp.int32(0), jnp.int32(_ZR * (_CG // 16)), initz,
                  jnp.int32(0))

    def inito(i, carry):
        r = i // i32(_CG // 16)
        k = i % i32(_CG // 16)
        ones_v[r, pl.ds(k * i32(16), 16)] = jnp.ones((16,), jnp.float32)
        return carry

    lax.fori_loop(jnp.int32(0), jnp.int32(_SCH * (_CG // 16)), inito,
                  jnp.int32(0))

    def _zero_own():
        for z in range(_RPT // _ZR):
            pltpu.sync_copy(
                zrow_v, acc_sh.at[pl.ds(rbase + i32(z * _ZR), _ZR)])

    def _edge_chunk_idx(c):
        eoff = pl.multiple_of(sid * i32(_EPT) + c * i32(_SCH), 8)
        pltpu.sync_copy(sidx_hbm.at[pl.ds(eoff, _SCH)], sidx_v)
        pltpu.sync_copy(oidx_hbm.at[pl.ds(eoff, _SCH)], oidx_v)
        return eoff

    _zero_own()
    plsc.subcore_barrier()

    for half in range(_NC):

        @pl.when(cid == half)
        def _half():
            cols = (0, _CG) if half == 0 else (2 * _CG, 3 * _CG)
            npass = len(cols) + (1 if half == 1 else 0)
            for p in range(npass):
                is_cnt = p >= len(cols)
                col = 0 if is_cnt else cols[p]

                def chunk(c, carry):
                    _edge_chunk_idx(c)
                    if not is_cnt:
                        eoff = pl.multiple_of(
                            sid * i32(_EPT) + c * i32(_SCH), 8)
                        pltpu.sync_copy(
                            ns_hbm.at[pl.ds(eoff, _SCH), pl.ds(col, _CG)],
                            srow_v)
                        pltpu.sync_copy(
                            no_hbm.at[pl.ds(eoff, _SCH), pl.ds(col, _CG)],
                            orow_v)
                        pltpu.sync_copy(srow_v, acc_sh.at[sidx_v], add=True)
                        pltpu.sync_copy(orow_v, acc_sh.at[oidx_v], add=True)
                    else:
                        pltpu.sync_copy(ones_v, acc_sh.at[sidx_v], add=True)
                        pltpu.sync_copy(ones_v, acc_sh.at[oidx_v], add=True)
                    return carry

                lax.fori_loop(jnp.int32(0), jnp.int32(_SN), chunk,
                              jnp.int32(0))
                plsc.subcore_barrier()
                if is_cnt:
                    pltpu.sync_copy(acc_sh.at[pl.ds(rbase, _RPT)],
                                    cnt_hbm.at[pl.ds(rbase, _RPT)])
                else:
                    pltpu.sync_copy(
                        acc_sh.at[pl.ds(rbase, _RPT)],
                        pooled_hbm.at[pl.ds(rbase, _RPT), pl.ds(col, _CG)])
                if p < npass - 1:
                    _zero_own()
                plsc.subcore_barrier()


def _z():
    return jnp.int32(0)


_BT = 512
_GT = _T // _BT


def _mlp_body(s_ref, p_ref, o_ref, was_ref, wap_ref, wao_ref, b1a_ref,
              wbs_ref, wbp_ref, wbo_ref, bbs_ref, bbp_ref, bbo_ref,
              ns_ref, np_ref, no_ref):
    f32 = jnp.float32
    s = s_ref[...].astype(jnp.bfloat16)
    p = p_ref[...].astype(jnp.bfloat16)
    o = o_ref[...].astype(jnp.bfloat16)
    h = jnp.dot(s, was_ref[...], preferred_element_type=f32)
    h += jnp.dot(p, wap_ref[...], preferred_element_type=f32)
    h += jnp.dot(o, wao_ref[...], preferred_element_type=f32)
    h = jnp.maximum(h + b1a_ref[...], 0.0).astype(jnp.bfloat16)
    ns_ref[...] = jnp.maximum(
        jnp.dot(h, wbs_ref[...], preferred_element_type=f32) + bbs_ref[...], 0.0)
    np_ref[...] = jnp.maximum(
        jnp.dot(h, wbp_ref[...], preferred_element_type=f32) + bbp_ref[...], 0.0)
    no_ref[...] = jnp.maximum(
        jnp.dot(h, wbo_ref[...], preferred_element_type=f32) + bbo_ref[...], 0.0)


_mlp_tc = pl.pallas_call(
    _mlp_body,
    grid=(_GT,),
    in_specs=[
        pl.BlockSpec((_BT, _D), lambda i: (i, _z())),
        pl.BlockSpec((_BT, _D), lambda i: (i, _z())),
        pl.BlockSpec((_BT, _D), lambda i: (i, _z())),
        pl.BlockSpec((_D, _H), lambda i: (_z(), _z())),
        pl.BlockSpec((_D, _H), lambda i: (_z(), _z())),
        pl.BlockSpec((_D, _H), lambda i: (_z(), _z())),
        pl.BlockSpec((1, _H), lambda i: (_z(), _z())),
        pl.BlockSpec((_H, _H), lambda i: (_z(), _z())),
        pl.BlockSpec((_H, _DOUT), lambda i: (_z(), _z())),
        pl.BlockSpec((_H, _H), lambda i: (_z(), _z())),
        pl.BlockSpec((1, _H), lambda i: (_z(), _z())),
        pl.BlockSpec((1, _DOUT), lambda i: (_z(), _z())),
        pl.BlockSpec((1, _H), lambda i: (_z(), _z())),
    ],
    out_specs=[
        pl.BlockSpec((_BT, _H), lambda i: (i, _z())),
        pl.BlockSpec((_BT, _DOUT), lambda i: (i, _z())),
        pl.BlockSpec((_BT, _H), lambda i: (i, _z())),
    ],
    out_shape=[
        jax.ShapeDtypeStruct((_T, _H), jnp.float32),
        jax.ShapeDtypeStruct((_T, _DOUT), jnp.float32),
        jax.ShapeDtypeStruct((_T, _H), jnp.float32),
    ],
    compiler_params=pltpu.CompilerParams(
        dimension_semantics=("arbitrary",)),
)


_BN = 1024
_GN2 = _OP // _BN


def _net2_body(pool_ref, cnt_ref, w2a_ref, b2a_ref, w2b_ref, b2b_ref, out_ref):
    f32 = jnp.float32
    cnt = jnp.maximum(cnt_ref[:, :1], 1.0)
    pool = (pool_ref[...] / cnt).astype(jnp.bfloat16)
    h = jnp.maximum(
        jnp.dot(pool, w2a_ref[...], preferred_element_type=f32) + b2a_ref[...],
        0.0).astype(jnp.bfloat16)
    out_ref[...] = jnp.maximum(
        jnp.dot(h, w2b_ref[...], preferred_element_type=f32) + b2b_ref[...],
        0.0)


_net2_tc = pl.pallas_call(
    _net2_body,
    grid=(_GN2,),
    in_specs=[
        pl.BlockSpec((_BN, _H), lambda i: (i, _z())),
        pl.BlockSpec((_BN, _CG), lambda i: (i, _z())),
        pl.BlockSpec((_H, _H), lambda i: (_z(), _z())),
        pl.BlockSpec((1, _H), lambda i: (_z(), _z())),
        pl.BlockSpec((_H, _DOUT), lambda i: (_z(), _z())),
        pl.BlockSpec((1, _DOUT), lambda i: (_z(), _z())),
    ],
    out_specs=pl.BlockSpec((_BN, _DOUT), lambda i: (i, _z())),
    out_shape=jax.ShapeDtypeStruct((_OP, _DOUT), jnp.float32),
    compiler_params=pltpu.CompilerParams(
        dimension_semantics=("arbitrary",)),
)


def kernel(obj_vecs, pred_vecs, edges, W1a, b1a, W1b, b1b, W2a, b2a, W2b, b2b):
    bf16 = jnp.bfloat16
    sidx = edges[:, 0].astype(jnp.int32)
    oidx = edges[:, 1].astype(jnp.int32)

    s_rows, o_rows = _gather_sc(obj_vecs, sidx, oidx)

    was = W1a[:, :_D].T.astype(bf16)
    wap = W1a[:, _D:2 * _D].T.astype(bf16)
    wao = W1a[:, 2 * _D:].T.astype(bf16)
    wbs = W1b[:_H].T.astype(bf16)
    wbp = W1b[_H:_H + _DOUT].T.astype(bf16)
    wbo = W1b[_H + _DOUT:].T.astype(bf16)

    new_s, new_p, new_o = _mlp_tc(
        s_rows, pred_vecs.astype(bf16), o_rows,
        was, wap, wao, b1a.reshape(1, _H).astype(jnp.float32),
        wbs, wbp, wbo,
        b1b[:_H].reshape(1, _H).astype(jnp.float32),
        b1b[_H:_H + _DOUT].reshape(1, _DOUT).astype(jnp.float32),
        b1b[_H + _DOUT:].reshape(1, _H).astype(jnp.float32))

    pooled, cnt = _scatter_sc(new_s, new_o, sidx, oidx)

    new_obj = _net2_tc(
        pooled, cnt, W2a.T.astype(bf16),
        b2a.reshape(1, _H).astype(jnp.float32), W2b.T.astype(bf16),
        b2b.reshape(1, _DOUT).astype(jnp.float32))

    return (new_obj[:_O].astype(jnp.float64), new_p.astype(jnp.float64))


# speedup vs baseline: 92.2441x; 1.0549x over previous
import functools

import jax
import jax.numpy as jnp
from jax import lax
from jax.experimental import pallas as pl
from jax.experimental.pallas import tpu as pltpu
from jax.experimental.pallas import tpu_sc as plsc

_O, _T, _D, _H, _DOUT = 10000, 320000, 128, 512, 128

_NC, _NS = 2, 16
_NW = _NC * _NS

_EPW = _T // _NW
_GCH = 400
_GN = _EPW // _GCH

_sc_mesh = plsc.VectorSubcoreMesh(core_axis_name="c", subcore_axis_name="s")


@functools.partial(
    pl.kernel,
    mesh=_sc_mesh,
    out_type=[
        jax.ShapeDtypeStruct((_T, _D), jnp.float32),
        jax.ShapeDtypeStruct((_T, _D), jnp.float32),
    ],
    scratch_types=[
        pltpu.VMEM((_GCH,), jnp.int32),
        pltpu.VMEM((_GCH,), jnp.int32),
        pltpu.VMEM((_GCH, _D), jnp.float32),
        pltpu.VMEM((_GCH, _D), jnp.float32),
        pltpu.SemaphoreType.DMA,
        pltpu.SemaphoreType.DMA,
    ],
)
def _gather_sc(obj_hbm, sidx_hbm, oidx_hbm, srows_hbm, orows_hbm,
               sidx_v, oidx_v, srow_v, orow_v, sem_s, sem_o):
    i32 = jnp.int32
    wid = lax.axis_index("s") * i32(_NC) + lax.axis_index("c")
    base = wid * i32(_EPW)

    def body(c, carry):
        off = pl.multiple_of(base + c * i32(_GCH), 8)
        pltpu.sync_copy(sidx_hbm.at[pl.ds(off, _GCH)], sidx_v)
        pltpu.sync_copy(oidx_hbm.at[pl.ds(off, _GCH)], oidx_v)
        cs = pltpu.async_copy(obj_hbm.at[sidx_v], srow_v, sem_s)
        co = pltpu.async_copy(obj_hbm.at[oidx_v], orow_v, sem_o)
        cs.wait()
        co.wait()
        pltpu.sync_copy(srow_v, srows_hbm.at[pl.ds(off, _GCH)])
        pltpu.sync_copy(orow_v, orows_hbm.at[pl.ds(off, _GCH)])
        return carry

    lax.fori_loop(jnp.int32(0), jnp.int32(_GN), body, jnp.int32(0))


_OP = 10240
_EPT = _T // _NS
_SCH = 80
_SN = _EPT // _SCH
_CG = 128
_RPT = _OP // _NS
_ZR = 40


@functools.partial(
    pl.kernel,
    mesh=_sc_mesh,
    out_type=[
        jax.ShapeDtypeStruct((_OP, _H), jnp.float32),
        jax.ShapeDtypeStruct((_OP, _CG), jnp.float32),
    ],
    scratch_types=[
        pltpu.VMEM_SHARED((_OP, _CG), jnp.float32),
        pltpu.VMEM((_SCH,), jnp.int32),
        pltpu.VMEM((_SCH,), jnp.int32),
        pltpu.VMEM((_SCH,), jnp.int32),
        pltpu.VMEM((_SCH,), jnp.int32),
        pltpu.VMEM((_SCH, _CG), jnp.float32),
        pltpu.VMEM((_SCH, _CG), jnp.float32),
        pltpu.VMEM((_SCH, _CG), jnp.float32),
        pltpu.VMEM((_SCH, _CG), jnp.float32),
        pltpu.VMEM((_ZR, _CG), jnp.float32),
        pltpu.SemaphoreType.DMA,
        pltpu.SemaphoreType.DMA,
        pltpu.SemaphoreType.DMA,
        pltpu.SemaphoreType.DMA,
        pltpu.SemaphoreType.DMA,
    ],
)
def _scatter_sc(ns_hbm, no_hbm, sidx_hbm, oidx_hbm, pooled_hbm, cnt_hbm,
                acc_sh, sidx0, sidx1, oidx0, oidx1, srow0, srow1, orow0,
                orow1, zrow_v, ldsem0, ldsem1, scsem0, scsem1, zsem):
    i32 = jnp.int32
    cid = lax.axis_index("c")
    sid = lax.axis_index("s")
    rbase = sid * i32(_RPT)
    sidx = (sidx0, sidx1)
    oidx = (oidx0, oidx1)
    srow = (srow0, srow1)
    orow = (orow0, orow1)
    ldsem = (ldsem0, ldsem1)
    scsem = (scsem0, scsem1)

    def initz(i, carry):
        r = i // i32(_CG // 16)
        k = i % i32(_CG // 16)
        zrow_v[r, pl.ds(k * i32(16), 16)] = jnp.zeros((16,), jnp.float32)
        return carry

    lax.fori_loop(jnp.int32(0), jnp.int32(_ZR * (_CG // 16)), initz,
                  jnp.int32(0))

    def _zero_own():
        for z in range(_RPT // _ZR):
            pltpu.async_copy(
                zrow_v, acc_sh.at[pl.ds(rbase + i32(z * _ZR), _ZR)], zsem)
        for z in range(_RPT // _ZR):
            pltpu.make_async_copy(
                zrow_v, acc_sh.at[pl.ds(rbase + i32(z * _ZR), _ZR)],
                zsem).wait()

    def _fill_ones():
        def body(i, carry):
            r = i // i32(_CG // 16)
            k = i % i32(_CG // 16)
            v = jnp.ones((16,), jnp.float32)
            srow0[r, pl.ds(k * i32(16), 16)] = v
            srow1[r, pl.ds(k * i32(16), 16)] = v
            orow0[r, pl.ds(k * i32(16), 16)] = v
            orow1[r, pl.ds(k * i32(16), 16)] = v
            return carry

        lax.fori_loop(jnp.int32(0), jnp.int32(_SCH * (_CG // 16)), body,
                      jnp.int32(0))

    def _eoff(c):
        return pl.multiple_of(sid * i32(_EPT) + c * i32(_SCH), 8)

    def run_pass(col, is_cnt):
        def load_copies(c, b):
            eoff = _eoff(c)
            out = [
                (sidx_hbm.at[pl.ds(eoff, _SCH)], sidx[b]),
                (oidx_hbm.at[pl.ds(eoff, _SCH)], oidx[b]),
            ]
            if not is_cnt:
                out += [
                    (ns_hbm.at[pl.ds(eoff, _SCH), pl.ds(col, _CG)], srow[b]),
                    (no_hbm.at[pl.ds(eoff, _SCH), pl.ds(col, _CG)], orow[b]),
                ]
            return out

        def issue_load(c, b):
            for s, d in load_copies(c, b):
                pltpu.async_copy(s, d, ldsem[b])

        def drain_load(c, b):
            for s, d in load_copies(c, b):
                pltpu.make_async_copy(s, d, ldsem[b]).wait()

        def issue_scatter(b):
            pltpu.async_copy(srow[b], acc_sh.at[sidx[b]], scsem[b], add=True)
            pltpu.async_copy(orow[b], acc_sh.at[oidx[b]], scsem[b], add=True)

        def drain_scatter(b):
            pltpu.make_async_copy(srow[b], acc_sh.at[sidx[b]],
                                  scsem[b]).wait()
            pltpu.make_async_copy(orow[b], acc_sh.at[oidx[b]],
                                  scsem[b]).wait()

        issue_load(jnp.int32(0), 0)

        def body2(c2, carry):
            for b in range(2):
                c = c2 * i32(2) + i32(b)
                drain_load(c, b)
                nb = 1 - b
                if b == 0:
                    @pl.when(c2 > 0)
                    def _():
                        drain_scatter(nb)
                    issue_load(c + i32(1), nb)
                else:
                    drain_scatter(nb)

                    @pl.when(c2 < i32(_SN // 2 - 1))
                    def _():
                        issue_load(c + i32(1), nb)
                issue_scatter(b)
            return carry

        lax.fori_loop(jnp.int32(0), jnp.int32(_SN // 2), body2, jnp.int32(0))
        drain_scatter(1)

    _zero_own()
    plsc.subcore_barrier()

    for half in range(_NC):

        @pl.when(cid == half)
        def _half():
            cols = (0, _CG) if half == 0 else (2 * _CG, 3 * _CG)
            for p, col in enumerate(cols):
                run_pass(col, False)
                plsc.subcore_barrier()
                pltpu.sync_copy(
                    acc_sh.at[pl.ds(rbase, _RPT)],
                    pooled_hbm.at[pl.ds(rbase, _RPT), pl.ds(col, _CG)])
                if p < len(cols) - 1 or half == 1:
                    _zero_own()
                plsc.subcore_barrier()
            if half == 1:
                _fill_ones()
                run_pass(0, True)
                plsc.subcore_barrier()
                pltpu.sync_copy(acc_sh.at[pl.ds(rbase, _RPT)],
                                cnt_hbm.at[pl.ds(rbase, _RPT)])
                plsc.subcore_barrier()


def _z():
    return jnp.int32(0)


_BT = 1280
_GT = _T // _BT


def _mlp_body(s_ref, p_ref, o_ref, was_ref, wap_ref, wao_ref, b1a_ref,
              wbs_ref, wbp_ref, wbo_ref, bbs_ref, bbp_ref, bbo_ref,
              ns_ref, np_ref, no_ref):
    f32 = jnp.float32
    s = s_ref[...].astype(jnp.bfloat16)
    p = p_ref[...].astype(jnp.bfloat16)
    o = o_ref[...].astype(jnp.bfloat16)
    h = jnp.dot(s, was_ref[...], preferred_element_type=f32)
    h += jnp.dot(p, wap_ref[...], preferred_element_type=f32)
    h += jnp.dot(o, wao_ref[...], preferred_element_type=f32)
    h = jnp.maximum(h + b1a_ref[...], 0.0).astype(jnp.bfloat16)
    ns_ref[...] = jnp.maximum(
        jnp.dot(h, wbs_ref[...], preferred_element_type=f32) + bbs_ref[...], 0.0)
    np_ref[...] = jnp.maximum(
        jnp.dot(h, wbp_ref[...], preferred_element_type=f32) + bbp_ref[...], 0.0)
    no_ref[...] = jnp.maximum(
        jnp.dot(h, wbo_ref[...], preferred_element_type=f32) + bbo_ref[...], 0.0)


_mlp_tc = pl.pallas_call(
    _mlp_body,
    grid=(_GT,),
    in_specs=[
        pl.BlockSpec((_BT, _D), lambda i: (i, _z())),
        pl.BlockSpec((_BT, _D), lambda i: (i, _z())),
        pl.BlockSpec((_BT, _D), lambda i: (i, _z())),
        pl.BlockSpec((_D, _H), lambda i: (_z(), _z())),
        pl.BlockSpec((_D, _H), lambda i: (_z(), _z())),
        pl.BlockSpec((_D, _H), lambda i: (_z(), _z())),
        pl.BlockSpec((1, _H), lambda i: (_z(), _z())),
        pl.BlockSpec((_H, _H), lambda i: (_z(), _z())),
        pl.BlockSpec((_H, _DOUT), lambda i: (_z(), _z())),
        pl.BlockSpec((_H, _H), lambda i: (_z(), _z())),
        pl.BlockSpec((1, _H), lambda i: (_z(), _z())),
        pl.BlockSpec((1, _DOUT), lambda i: (_z(), _z())),
        pl.BlockSpec((1, _H), lambda i: (_z(), _z())),
    ],
    out_specs=[
        pl.BlockSpec((_BT, _H), lambda i: (i, _z())),
        pl.BlockSpec((_BT, _DOUT), lambda i: (i, _z())),
        pl.BlockSpec((_BT, _H), lambda i: (i, _z())),
    ],
    out_shape=[
        jax.ShapeDtypeStruct((_T, _H), jnp.float32),
        jax.ShapeDtypeStruct((_T, _DOUT), jnp.float32),
        jax.ShapeDtypeStruct((_T, _H), jnp.float32),
    ],
    compiler_params=pltpu.CompilerParams(
        dimension_semantics=("arbitrary",)),
)


_BN = 1024
_GN2 = _OP // _BN


def _net2_body(pool_ref, cnt_ref, w2a_ref, b2a_ref, w2b_ref, b2b_ref, out_ref):
    f32 = jnp.float32
    cnt = jnp.maximum(cnt_ref[:, :1], 1.0)
    pool = (pool_ref[...] / cnt).astype(jnp.bfloat16)
    h = jnp.maximum(
        jnp.dot(pool, w2a_ref[...], preferred_element_type=f32) + b2a_ref[...],
        0.0).astype(jnp.bfloat16)
    out_ref[...] = jnp.maximum(
        jnp.dot(h, w2b_ref[...], preferred_element_type=f32) + b2b_ref[...],
        0.0)


_net2_tc = pl.pallas_call(
    _net2_body,
    grid=(_GN2,),
    in_specs=[
        pl.BlockSpec((_BN, _H), lambda i: (i, _z())),
        pl.BlockSpec((_BN, _CG), lambda i: (i, _z())),
        pl.BlockSpec((_H, _H), lambda i: (_z(), _z())),
        pl.BlockSpec((1, _H), lambda i: (_z(), _z())),
        pl.BlockSpec((_H, _DOUT), lambda i: (_z(), _z())),
        pl.BlockSpec((1, _DOUT), lambda i: (_z(), _z())),
    ],
    out_specs=pl.BlockSpec((_BN, _DOUT), lambda i: (i, _z())),
    out_shape=jax.ShapeDtypeStruct((_OP, _DOUT), jnp.float32),
    compiler_params=pltpu.CompilerParams(
        dimension_semantics=("arbitrary",)),
)


def kernel(obj_vecs, pred_vecs, edges, W1a, b1a, W1b, b1b, W2a, b2a, W2b, b2b):
    bf16 = jnp.bfloat16
    sidx = edges[:, 0].astype(jnp.int32)
    oidx = edges[:, 1].astype(jnp.int32)

    s_rows, o_rows = _gather_sc(obj_vecs, sidx, oidx)

    was = W1a[:, :_D].T.astype(bf16)
    wap = W1a[:, _D:2 * _D].T.astype(bf16)
    wao = W1a[:, 2 * _D:].T.astype(bf16)
    wbs = W1b[:_H].T.astype(bf16)
    wbp = W1b[_H:_H + _DOUT].T.astype(bf16)
    wbo = W1b[_H + _DOUT:].T.astype(bf16)

    new_s, new_p, new_o = _mlp_tc(
        s_rows, pred_vecs.astype(bf16), o_rows,
        was, wap, wao, b1a.reshape(1, _H).astype(jnp.float32),
        wbs, wbp, wbo,
        b1b[:_H].reshape(1, _H).astype(jnp.float32),
        b1b[_H:_H + _DOUT].reshape(1, _DOUT).astype(jnp.float32),
        b1b[_H + _DOUT:].reshape(1, _H).astype(jnp.float32))

    pooled, cnt = _scatter_sc(new_s, new_o, sidx, oidx)

    new_obj = _net2_tc(
        pooled, cnt, W2a.T.astype(bf16),
        b2a.reshape(1, _H).astype(jnp.float32), W2b.T.astype(bf16),
        b2b.reshape(1, _DOUT).astype(jnp.float32))

    return (new_obj[:_O].astype(jnp.float64), new_p.astype(jnp.float64))


# speedup vs baseline: 93.7816x; 1.0167x over previous
import functools

import jax
import jax.numpy as jnp
from jax import lax
from jax.experimental import pallas as pl
from jax.experimental.pallas import tpu as pltpu
from jax.experimental.pallas import tpu_sc as plsc

_O, _T, _D, _H, _DOUT = 10000, 320000, 128, 512, 128

_NC, _NS = 2, 16
_NW = _NC * _NS

_EPW = _T // _NW
_GCH = 400
_GN = _EPW // _GCH

_sc_mesh = plsc.VectorSubcoreMesh(core_axis_name="c", subcore_axis_name="s")


@functools.partial(
    pl.kernel,
    mesh=_sc_mesh,
    out_type=[
        jax.ShapeDtypeStruct((_T, _D), jnp.float32),
        jax.ShapeDtypeStruct((_T, _D), jnp.float32),
    ],
    scratch_types=[
        pltpu.VMEM((_GCH,), jnp.int32),
        pltpu.VMEM((_GCH,), jnp.int32),
        pltpu.VMEM((_GCH, _D), jnp.float32),
        pltpu.VMEM((_GCH, _D), jnp.float32),
        pltpu.SemaphoreType.DMA,
        pltpu.SemaphoreType.DMA,
    ],
)
def _gather_sc(obj_hbm, sidx_hbm, oidx_hbm, srows_hbm, orows_hbm,
               sidx_v, oidx_v, srow_v, orow_v, sem_s, sem_o):
    i32 = jnp.int32
    wid = lax.axis_index("s") * i32(_NC) + lax.axis_index("c")
    base = wid * i32(_EPW)

    def body(c, carry):
        off = pl.multiple_of(base + c * i32(_GCH), 8)
        pltpu.sync_copy(sidx_hbm.at[pl.ds(off, _GCH)], sidx_v)
        pltpu.sync_copy(oidx_hbm.at[pl.ds(off, _GCH)], oidx_v)
        cs = pltpu.async_copy(obj_hbm.at[sidx_v], srow_v, sem_s)
        co = pltpu.async_copy(obj_hbm.at[oidx_v], orow_v, sem_o)
        cs.wait()
        co.wait()
        pltpu.sync_copy(srow_v, srows_hbm.at[pl.ds(off, _GCH)])
        pltpu.sync_copy(orow_v, orows_hbm.at[pl.ds(off, _GCH)])
        return carry

    lax.fori_loop(jnp.int32(0), jnp.int32(_GN), body, jnp.int32(0))


_OP = 10240
_EPT = _T // _NS
_SCH = 80
_SN = _EPT // _SCH
_CG = 128
_RPT = _OP // _NS
_ZR = 40


@functools.partial(
    pl.kernel,
    mesh=_sc_mesh,
    out_type=[
        jax.ShapeDtypeStruct((_OP, _H), jnp.float32),
        jax.ShapeDtypeStruct((_OP, _CG), jnp.float32),
    ],
    scratch_types=[
        pltpu.VMEM_SHARED((_OP, _CG), jnp.float32),
        pltpu.VMEM((_SCH,), jnp.int32),
        pltpu.VMEM((_SCH,), jnp.int32),
        pltpu.VMEM((_SCH,), jnp.int32),
        pltpu.VMEM((_SCH,), jnp.int32),
        pltpu.VMEM((_SCH, _CG), jnp.float32),
        pltpu.VMEM((_SCH, _CG), jnp.float32),
        pltpu.VMEM((_SCH, _CG), jnp.float32),
        pltpu.VMEM((_SCH, _CG), jnp.float32),
        pltpu.VMEM((_ZR, _CG), jnp.float32),
        pltpu.SemaphoreType.DMA,
        pltpu.SemaphoreType.DMA,
        pltpu.SemaphoreType.DMA,
        pltpu.SemaphoreType.DMA,
        pltpu.SemaphoreType.DMA,
    ],
)
def _scatter_sc(ns_hbm, no_hbm, sidx_hbm, oidx_hbm, pooled_hbm, cnt_hbm,
                acc_sh, sidx0, sidx1, oidx0, oidx1, srow0, srow1, orow0,
                orow1, zrow_v, ldsem0, ldsem1, scsem0, scsem1, zsem):
    i32 = jnp.int32
    cid = lax.axis_index("c")
    sid = lax.axis_index("s")
    rbase = sid * i32(_RPT)
    sidx = (sidx0, sidx1)
    oidx = (oidx0, oidx1)
    srow = (srow0, srow1)
    orow = (orow0, orow1)
    ldsem = (ldsem0, ldsem1)
    scsem = (scsem0, scsem1)

    def initz(i, carry):
        r = i // i32(_CG // 16)
        k = i % i32(_CG // 16)
        zrow_v[r, pl.ds(k * i32(16), 16)] = jnp.zeros((16,), jnp.float32)
        return carry

    lax.fori_loop(jnp.int32(0), jnp.int32(_ZR * (_CG // 16)), initz,
                  jnp.int32(0))

    def _zero_own():
        for z in range(_RPT // _ZR):
            pltpu.async_copy(
                zrow_v, acc_sh.at[pl.ds(rbase + i32(z * _ZR), _ZR)], zsem)
        for z in range(_RPT // _ZR):
            pltpu.make_async_copy(
                zrow_v, acc_sh.at[pl.ds(rbase + i32(z * _ZR), _ZR)],
                zsem).wait()

    def _fill_ones():
        def body(i, carry):
            r = i // i32(_CG // 16)
            k = i % i32(_CG // 16)
            v = jnp.ones((16,), jnp.float32)
            srow0[r, pl.ds(k * i32(16), 16)] = v
            srow1[r, pl.ds(k * i32(16), 16)] = v
            orow0[r, pl.ds(k * i32(16), 16)] = v
            orow1[r, pl.ds(k * i32(16), 16)] = v
            return carry

        lax.fori_loop(jnp.int32(0), jnp.int32(_SCH * (_CG // 16)), body,
                      jnp.int32(0))

    def _eoff(c):
        return pl.multiple_of(sid * i32(_EPT) + c * i32(_SCH), 8)

    def run_pass(col, is_cnt):
        def load_copies(c, b):
            eoff = _eoff(c)
            out = [
                (sidx_hbm.at[pl.ds(eoff, _SCH)], sidx[b]),
                (oidx_hbm.at[pl.ds(eoff, _SCH)], oidx[b]),
            ]
            if not is_cnt:
                out += [
                    (ns_hbm.at[pl.ds(eoff, _SCH), pl.ds(col, _CG)], srow[b]),
                    (no_hbm.at[pl.ds(eoff, _SCH), pl.ds(col, _CG)], orow[b]),
                ]
            return out

        def issue_load(c, b):
            for s, d in load_copies(c, b):
                pltpu.async_copy(s, d, ldsem[b])

        def drain_load(c, b):
            for s, d in load_copies(c, b):
                pltpu.make_async_copy(s, d, ldsem[b]).wait()

        def issue_scatter(b):
            pltpu.async_copy(srow[b], acc_sh.at[sidx[b]], scsem[b], add=True)
            pltpu.async_copy(orow[b], acc_sh.at[oidx[b]], scsem[b], add=True)

        def drain_scatter(b):
            pltpu.make_async_copy(srow[b], acc_sh.at[sidx[b]],
                                  scsem[b]).wait()
            pltpu.make_async_copy(orow[b], acc_sh.at[oidx[b]],
                                  scsem[b]).wait()

        issue_load(jnp.int32(0), 0)

        def body2(c2, carry):
            for b in range(2):
                c = c2 * i32(2) + i32(b)
                drain_load(c, b)
                nb = 1 - b
                if b == 0:
                    @pl.when(c2 > 0)
                    def _():
                        drain_scatter(nb)
                    issue_load(c + i32(1), nb)
                else:
                    drain_scatter(nb)

                    @pl.when(c2 < i32(_SN // 2 - 1))
                    def _():
                        issue_load(c + i32(1), nb)
                issue_scatter(b)
            return carry

        lax.fori_loop(jnp.int32(0), jnp.int32(_SN // 2), body2, jnp.int32(0))
        drain_scatter(1)

    _zero_own()
    plsc.subcore_barrier()

    for half in range(_NC):

        @pl.when(cid == half)
        def _half():
            cols = (0, _CG) if half == 0 else (2 * _CG, 3 * _CG)
            for p, col in enumerate(cols):
                run_pass(col, False)
                plsc.subcore_barrier()
                pltpu.sync_copy(
                    acc_sh.at[pl.ds(rbase, _RPT)],
                    pooled_hbm.at[pl.ds(rbase, _RPT), pl.ds(col, _CG)])
                if p < len(cols) - 1 or half == 1:
                    _zero_own()
                plsc.subcore_barrier()
            if half == 1:
                _fill_ones()
                run_pass(0, True)
                plsc.subcore_barrier()
                pltpu.sync_copy(acc_sh.at[pl.ds(rbase, _RPT)],
                                cnt_hbm.at[pl.ds(rbase, _RPT)])
                plsc.subcore_barrier()


def _z():
    return jnp.int32(0)


_BT = 1280
_GT = _T // _BT


def _mlp_body(s_ref, p_ref, o_ref, was_ref, wap_ref, wao_ref, b1a_ref,
              wbs_ref, wbp_ref, wbo_ref, bbs_ref, bbp_ref, bbo_ref,
              ns_ref, np_ref, no_ref):
    f32 = jnp.float32
    dn = (((1,), (1,)), ((), ()))
    s = s_ref[...].astype(jnp.bfloat16)
    p = p_ref[...].astype(jnp.bfloat16)
    o = o_ref[...].astype(jnp.bfloat16)
    h = lax.dot_general(s, was_ref[...], dn, preferred_element_type=f32)
    h += lax.dot_general(p, wap_ref[...], dn, preferred_element_type=f32)
    h += lax.dot_general(o, wao_ref[...], dn, preferred_element_type=f32)
    h = jnp.maximum(h + b1a_ref[...], 0.0).astype(jnp.bfloat16)
    ns_ref[...] = jnp.maximum(
        lax.dot_general(h, wbs_ref[...], dn, preferred_element_type=f32)
        + bbs_ref[...], 0.0)
    np_ref[...] = jnp.maximum(
        lax.dot_general(h, wbp_ref[...], dn, preferred_element_type=f32)
        + bbp_ref[...], 0.0)
    no_ref[...] = jnp.maximum(
        lax.dot_general(h, wbo_ref[...], dn, preferred_element_type=f32)
        + bbo_ref[...], 0.0)


_mlp_tc = pl.pallas_call(
    _mlp_body,
    grid=(_GT,),
    in_specs=[
        pl.BlockSpec((_BT, _D), lambda i: (i, _z())),
        pl.BlockSpec((_BT, _D), lambda i: (i, _z())),
        pl.BlockSpec((_BT, _D), lambda i: (i, _z())),
        pl.BlockSpec((_H, _D), lambda i: (_z(), _z())),
        pl.BlockSpec((_H, _D), lambda i: (_z(), _z())),
        pl.BlockSpec((_H, _D), lambda i: (_z(), _z())),
        pl.BlockSpec((1, _H), lambda i: (_z(), _z())),
        pl.BlockSpec((_H, _H), lambda i: (_z(), _z())),
        pl.BlockSpec((_DOUT, _H), lambda i: (_z(), _z())),
        pl.BlockSpec((_H, _H), lambda i: (_z(), _z())),
        pl.BlockSpec((1, _H), lambda i: (_z(), _z())),
        pl.BlockSpec((1, _DOUT), lambda i: (_z(), _z())),
        pl.BlockSpec((1, _H), lambda i: (_z(), _z())),
    ],
    out_specs=[
        pl.BlockSpec((_BT, _H), lambda i: (i, _z())),
        pl.BlockSpec((_BT, _DOUT), lambda i: (i, _z())),
        pl.BlockSpec((_BT, _H), lambda i: (i, _z())),
    ],
    out_shape=[
        jax.ShapeDtypeStruct((_T, _H), jnp.float32),
        jax.ShapeDtypeStruct((_T, _DOUT), jnp.float32),
        jax.ShapeDtypeStruct((_T, _H), jnp.float32),
    ],
    compiler_params=pltpu.CompilerParams(
        dimension_semantics=("arbitrary",)),
)


_BN = 1024
_GN2 = _OP // _BN


def _net2_body(pool_ref, cnt_ref, w2a_ref, b2a_ref, w2b_ref, b2b_ref, out_ref):
    f32 = jnp.float32
    dn = (((1,), (1,)), ((), ()))
    cnt = jnp.maximum(cnt_ref[:, :1], 1.0)
    pool = (pool_ref[...] / cnt).astype(jnp.bfloat16)
    h = jnp.maximum(
        lax.dot_general(pool, w2a_ref[...], dn, preferred_element_type=f32)
        + b2a_ref[...], 0.0).astype(jnp.bfloat16)
    out_ref[...] = jnp.maximum(
        lax.dot_general(h, w2b_ref[...], dn, preferred_element_type=f32)
        + b2b_ref[...], 0.0)


_net2_tc = pl.pallas_call(
    _net2_body,
    grid=(_GN2,),
    in_specs=[
        pl.BlockSpec((_BN, _H), lambda i: (i, _z())),
        pl.BlockSpec((_BN, _CG), lambda i: (i, _z())),
        pl.BlockSpec((_H, _H), lambda i: (_z(), _z())),
        pl.BlockSpec((1, _H), lambda i: (_z(), _z())),
        pl.BlockSpec((_DOUT, _H), lambda i: (_z(), _z())),
        pl.BlockSpec((1, _DOUT), lambda i: (_z(), _z())),
    ],
    out_specs=pl.BlockSpec((_BN, _DOUT), lambda i: (i, _z())),
    out_shape=jax.ShapeDtypeStruct((_OP, _DOUT), jnp.float32),
    compiler_params=pltpu.CompilerParams(
        dimension_semantics=("arbitrary",)),
)


def kernel(obj_vecs, pred_vecs, edges, W1a, b1a, W1b, b1b, W2a, b2a, W2b, b2b):
    bf16 = jnp.bfloat16
    sidx = edges[:, 0].astype(jnp.int32)
    oidx = edges[:, 1].astype(jnp.int32)

    s_rows, o_rows = _gather_sc(obj_vecs, sidx, oidx)

    w1ab = W1a.astype(bf16)
    w1bb = W1b.astype(bf16)
    was = w1ab[:, :_D]
    wap = w1ab[:, _D:2 * _D]
    wao = w1ab[:, 2 * _D:]
    wbs = w1bb[:_H]
    wbp = w1bb[_H:_H + _DOUT]
    wbo = w1bb[_H + _DOUT:]

    new_s, new_p, new_o = _mlp_tc(
        s_rows, pred_vecs, o_rows,
        was, wap, wao, b1a.reshape(1, _H).astype(jnp.float32),
        wbs, wbp, wbo,
        b1b[:_H].reshape(1, _H).astype(jnp.float32),
        b1b[_H:_H + _DOUT].reshape(1, _DOUT).astype(jnp.float32),
        b1b[_H + _DOUT:].reshape(1, _H).astype(jnp.float32))

    pooled, cnt = _scatter_sc(new_s, new_o, sidx, oidx)

    new_obj = _net2_tc(
        pooled, cnt, W2a.astype(bf16),
        b2a.reshape(1, _H).astype(jnp.float32), W2b.astype(bf16),
        b2b.reshape(1, _DOUT).astype(jnp.float32))

    return (new_obj[:_O].astype(jnp.float64), new_p.astype(jnp.float64))
